# baseline (speedup 1.0000x reference)
import functools

import jax
import jax.numpy as jnp
from jax import lax
from jax.experimental import pallas as pl
from jax.experimental.pallas import tpu as pltpu
from jax.experimental.pallas import tpu_sc as plsc

D_MODEL = 64
N_OBS = 200
N_FEAT = 128
NJ = D_MODEL // 16

NUM_CORES = 2
NUM_SUBCORES = 16
NW = NUM_CORES * NUM_SUBCORES

BLK = 640


@functools.lru_cache(maxsize=None)
def _build(T: int):
    per_w = T // NW
    steps = per_w // BLK
    assert per_w % BLK == 0 and steps % 2 == 0

    mesh = plsc.VectorSubcoreMesh(
        core_axis_name="c", subcore_axis_name="s",
        num_cores=NUM_CORES, num_subcores=NUM_SUBCORES)

    @functools.partial(
        pl.kernel,
        out_type=jax.ShapeDtypeStruct((T * D_MODEL,), jnp.float32),
        mesh=mesh,
        compiler_params=pltpu.CompilerParams(needs_layout_passes=False),
        scratch_types=[
            pltpu.VMEM((N_OBS * D_MODEL,), jnp.float32),
            pltpu.VMEM((N_FEAT * D_MODEL,), jnp.float32),
            pltpu.VMEM((D_MODEL,), jnp.float32),
            pltpu.VMEM((D_MODEL,), jnp.float32),
            pltpu.VMEM((2 * BLK,), jnp.int32),
            pltpu.VMEM((2 * BLK,), jnp.int32),
            pltpu.VMEM((2 * BLK,), jnp.float32),
            pltpu.VMEM((2 * BLK * D_MODEL,), jnp.float32),
            pltpu.VMEM((D_MODEL * 16,), jnp.float32),
            pltpu.SemaphoreType.DMA,
            pltpu.SemaphoreType.DMA,
            pltpu.SemaphoreType.DMA,
            pltpu.SemaphoreType.DMA,
        ],
    )
    def embed(val_h, obs_h, feat_h, tab_obs_h, tab_feat_h, w_h, bias_h,
              out_h, tab_o, tab_f, w_v, b_v, obs_v, feat_v, val_v, out_v,
              wsplat_v, sem_in0, sem_in1, sem_out0, sem_out1):
        wid = lax.axis_index("s") * NUM_CORES + lax.axis_index("c")
        base = wid * per_w
        sems_in = (sem_in0, sem_in1)
        sems_out = (sem_out0, sem_out1)

        pltpu.sync_copy(tab_obs_h, tab_o)
        pltpu.sync_copy(tab_feat_h, tab_f)
        pltpu.sync_copy(w_h, w_v)
        pltpu.sync_copy(bias_h, b_v)

        w_regs = [w_v[pl.ds(16 * j, 16)] for j in range(NJ)]
        b_regs = [b_v[pl.ds(16 * j, 16)] for j in range(NJ)]

        def bias_body(r, c):
            for j in range(NJ):
                off = r * D_MODEL + 16 * j
                tab_o[pl.ds(off, 16)] = tab_o[pl.ds(off, 16)] + b_regs[j]
            return c

        lax.fori_loop(0, N_OBS, bias_body, 0)

        for j in range(NJ):
            for l in range(16):
                wsplat_v[pl.ds((16 * j + l) * 16, 16)] = jnp.full(
                    (16,), w_regs[j][l], jnp.float32)

        iota16 = lax.iota(jnp.int32, 16)

        def in_copies(s, b):
            row0 = base + s * BLK
            return (
                pltpu.make_async_copy(obs_h.at[pl.ds(row0, BLK)],
                                      obs_v.at[pl.ds(b * BLK, BLK)],
                                      sems_in[b]),
                pltpu.make_async_copy(feat_h.at[pl.ds(row0, BLK)],
                                      feat_v.at[pl.ds(b * BLK, BLK)],
                                      sems_in[b]),
                pltpu.make_async_copy(val_h.at[pl.ds(row0, BLK)],
                                      val_v.at[pl.ds(b * BLK, BLK)],
                                      sems_in[b]),
            )

        def out_copy(s, b):
            row0 = base + s * BLK
            return pltpu.make_async_copy(
                out_v.at[pl.ds(b * BLK * D_MODEL, BLK * D_MODEL)],
                out_h.at[pl.ds(row0 * D_MODEL, BLK * D_MODEL)],
                sems_out[b])

        for c in in_copies(0, 0):
            c.start()

        def pair_body(g, carry):
            for b in range(2):
                s = g * 2 + b

                @pl.when(s + 1 < steps)
                def _():
                    for c in in_copies(s + 1, 1 - b):
                        c.start()

                for c in in_copies(s, b):
                    c.wait()

                @pl.when(s >= 2)
                def _():
                    out_copy(s - 2, b).wait()

                def grp_body(gi, carry2):
                    t0b = b * BLK + gi * 16
                    o16 = obs_v[pl.ds(t0b, 16)] * D_MODEL
                    f16 = feat_v[pl.ds(t0b, 16)] * D_MODEL
                    v16 = val_v[pl.ds(t0b, 16)]
                    dvec = iota16 * D_MODEL + t0b * D_MODEL
                    lo = plsc.load_gather(tab_o, [o16])
                    lf = plsc.load_gather(tab_f, [f16])
                    vw = v16 * wsplat_v[pl.ds(0, 16)]
                    for d in range(D_MODEL):
                        if d + 1 < D_MODEL:
                            lo_n = plsc.load_gather(tab_o, [o16 + (d + 1)])
                            lf_n = plsc.load_gather(tab_f, [f16 + (d + 1)])
                            vw_n = v16 * wsplat_v[pl.ds((d + 1) * 16, 16)]
                        acc = lo + lf + vw
                        plsc.store_scatter(out_v, [dvec + d], acc)
                        if d + 1 < D_MODEL:
                            lo, lf, vw = lo_n, lf_n, vw_n
                    return carry2

                lax.fori_loop(0, BLK // 16, grp_body, 0)
                out_copy(s, b).start()
            return carry

        lax.fori_loop(0, steps // 2, pair_body, 0)
        out_copy(steps - 2, 0).wait()
        out_copy(steps - 1, 1).wait()

    return embed


def kernel(val, obs_idx, feat_idx, W_val, b_val, emb_obs, emb_feat):
    B, L, _ = val.shape
    T = B * L
    val_f = val.reshape(T).astype(jnp.float32)
    obs_f = obs_idx.reshape(T).astype(jnp.int32)
    feat_f = feat_idx.reshape(T).astype(jnp.int32)
    w_f = W_val.reshape(D_MODEL).astype(jnp.float32)
    b_f = b_val.reshape(D_MODEL).astype(jnp.float32)
    out = _build(T)(val_f, obs_f, feat_f,
                    emb_obs.astype(jnp.float32).reshape(N_OBS * D_MODEL),
                    emb_feat.astype(jnp.float32).reshape(N_FEAT * D_MODEL),
                    w_f, b_f)
    return out.reshape(B, L, D_MODEL)


# speedup vs baseline: 4.1585x; 4.1585x over previous
import functools

import jax
import jax.numpy as jnp
from jax import lax
from jax.experimental import pallas as pl
from jax.experimental.pallas import tpu as pltpu
from jax.experimental.pallas import tpu_sc as plsc

D_MODEL = 64
N_OBS = 200
N_FEAT = 128
NJ = D_MODEL // 16

NUM_CORES = 2
NUM_SUBCORES = 16
NW = NUM_CORES * NUM_SUBCORES

BLK = 640


@functools.lru_cache(maxsize=None)
def _build(T: int):
    per_w = T // NW
    steps = per_w // BLK
    assert per_w % BLK == 0 and steps % 2 == 0

    mesh = plsc.VectorSubcoreMesh(
        core_axis_name="c", subcore_axis_name="s",
        num_cores=NUM_CORES, num_subcores=NUM_SUBCORES)

    @functools.partial(
        pl.kernel,
        out_type=jax.ShapeDtypeStruct((T * D_MODEL,), jnp.float32),
        mesh=mesh,
        compiler_params=pltpu.CompilerParams(needs_layout_passes=False),
        scratch_types=[
            pltpu.VMEM((N_OBS * D_MODEL,), jnp.float32),
            pltpu.VMEM((N_FEAT * D_MODEL,), jnp.float32),
            pltpu.VMEM((D_MODEL,), jnp.float32),
            pltpu.VMEM((D_MODEL,), jnp.float32),
            pltpu.VMEM((2 * BLK,), jnp.int32),
            pltpu.VMEM((2 * BLK,), jnp.int32),
            pltpu.VMEM((2 * BLK,), jnp.float32),
            pltpu.VMEM((2 * BLK * D_MODEL,), jnp.float32),
            pltpu.VMEM((D_MODEL * 16,), jnp.float32),
            pltpu.SemaphoreType.DMA,
            pltpu.SemaphoreType.DMA,
            pltpu.SemaphoreType.DMA,
            pltpu.SemaphoreType.DMA,
        ],
    )
    def embed(val_h, obs_h, feat_h, tab_obs_h, tab_feat_h, w_h, bias_h,
              out_h, tab_o, tab_f, w_v, b_v, obs_v, feat_v, val_v, out_v,
              wsplat_v, sem_in0, sem_in1, sem_out0, sem_out1):
        wid = lax.axis_index("s") * NUM_CORES + lax.axis_index("c")
        base = wid * per_w
        sems_in = (sem_in0, sem_in1)
        sems_out = (sem_out0, sem_out1)

        pltpu.sync_copy(tab_obs_h, tab_o)
        pltpu.sync_copy(tab_feat_h, tab_f)
        pltpu.sync_copy(w_h, w_v)
        pltpu.sync_copy(bias_h, b_v)

        w_regs = [w_v[pl.ds(16 * j, 16)] for j in range(NJ)]
        b_regs = [b_v[pl.ds(16 * j, 16)] for j in range(NJ)]

        def bias_body(r, c):
            for j in range(NJ):
                off = r * D_MODEL + 16 * j
                tab_o[pl.ds(off, 16)] = tab_o[pl.ds(off, 16)] + b_regs[j]
            return c

        lax.fori_loop(0, N_OBS, bias_body, 0)

        for j in range(NJ):
            for l in range(16):
                wsplat_v[pl.ds((16 * j + l) * 16, 16)] = jnp.full(
                    (16,), w_regs[j][l], jnp.float32)

        iota16 = lax.iota(jnp.int32, 16)

        def in_copies(s, b):
            row0 = base + s * BLK
            return (
                pltpu.make_async_copy(obs_h.at[pl.ds(row0, BLK)],
                                      obs_v.at[pl.ds(b * BLK, BLK)],
                                      sems_in[b]),
                pltpu.make_async_copy(feat_h.at[pl.ds(row0, BLK)],
                                      feat_v.at[pl.ds(b * BLK, BLK)],
                                      sems_in[b]),
                pltpu.make_async_copy(val_h.at[pl.ds(row0, BLK)],
                                      val_v.at[pl.ds(b * BLK, BLK)],
                                      sems_in[b]),
            )

        def out_copy(s, b):
            row0 = base + s * BLK
            return pltpu.make_async_copy(
                out_v.at[pl.ds(b * BLK * D_MODEL, BLK * D_MODEL)],
                out_h.at[pl.ds(row0 * D_MODEL, BLK * D_MODEL)],
                sems_out[b])

        for c in in_copies(0, 0):
            c.start()

        def pair_body(g, carry):
            for b in range(2):
                s = g * 2 + b

                @pl.when(s + 1 < steps)
                def _():
                    for c in in_copies(s + 1, 1 - b):
                        c.start()

                for c in in_copies(s, b):
                    c.wait()

                @pl.when(s >= 2)
                def _():
                    out_copy(s - 2, b).wait()

                def grp_body(gi, carry2):
                    t0b = b * BLK + gi * 16
                    o16 = obs_v[pl.ds(t0b, 16)] * D_MODEL
                    f16 = feat_v[pl.ds(t0b, 16)] * D_MODEL
                    v16 = val_v[pl.ds(t0b, 16)]
                    dst0 = t0b * D_MODEL

                    def load_tok(ob, fb):
                        return (
                            [tab_o[pl.ds(ob + 16 * j, 16)] for j in range(NJ)],
                            [tab_f[pl.ds(fb + 16 * j, 16)] for j in range(NJ)],
                        )

                    PF = 2
                    obase = [None] * 16
                    fbase = [None] * 16
                    for k in range(PF):
                        obase[k] = o16[k]
                        fbase[k] = f16[k]
                    ro, rf = load_tok(obase[0], fbase[0])
                    for k in range(16):
                        if k + PF < 16:
                            obase[k + PF] = o16[k + PF]
                            fbase[k + PF] = f16[k + PF]
                        if k + 1 < 16:
                            ro_n, rf_n = load_tok(obase[k + 1], fbase[k + 1])
                        vb = v16[k]
                        dst = dst0 + D_MODEL * k
                        for j in range(NJ):
                            out_v[pl.ds(dst + 16 * j, 16)] = (
                                ro[j] + rf[j] + vb * w_regs[j])
                        if k + 1 < 16:
                            ro, rf = ro_n, rf_n
                    return carry2

                lax.fori_loop(0, BLK // 16, grp_body, 0)
                out_copy(s, b).start()
            return carry

        lax.fori_loop(0, steps // 2, pair_body, 0)
        out_copy(steps - 2, 0).wait()
        out_copy(steps - 1, 1).wait()

    return embed


def kernel(val, obs_idx, feat_idx, W_val, b_val, emb_obs, emb_feat):
    B, L, _ = val.shape
    T = B * L
    val_f = val.reshape(T).astype(jnp.float32)
    obs_f = obs_idx.reshape(T).astype(jnp.int32)
    feat_f = feat_idx.reshape(T).astype(jnp.int32)
    w_f = W_val.reshape(D_MODEL).astype(jnp.float32)
    b_f = b_val.reshape(D_MODEL).astype(jnp.float32)
    out = _build(T)(val_f, obs_f, feat_f,
                    emb_obs.astype(jnp.float32).reshape(N_OBS * D_MODEL),
                    emb_feat.astype(jnp.float32).reshape(N_FEAT * D_MODEL),
                    w_f, b_f)
    return out.reshape(B, L, D_MODEL)


# speedup vs baseline: 5.7066x; 1.3723x over previous
import functools

import jax
import jax.numpy as jnp
from jax import lax
from jax.experimental import pallas as pl
from jax.experimental.pallas import tpu as pltpu
from jax.experimental.pallas import tpu_sc as plsc

D_MODEL = 64
N_OBS = 200
N_FEAT = 128
NJ = D_MODEL // 16

NUM_CORES = 2
NUM_SUBCORES = 16
NW = NUM_CORES * NUM_SUBCORES

BLK = 200
BLKP = 208


@functools.lru_cache(maxsize=None)
def _build(B: int, L: int):
    T = B * L
    rows_per_w = B // NW
    per_w = rows_per_w * L
    steps = rows_per_w
    assert L == BLK and steps % 2 == 0

    mesh = plsc.VectorSubcoreMesh(
        core_axis_name="c", subcore_axis_name="s",
        num_cores=NUM_CORES, num_subcores=NUM_SUBCORES)

    @functools.partial(
        pl.kernel,
        out_type=jax.ShapeDtypeStruct((B, L, D_MODEL), jnp.float32),
        mesh=mesh,
        compiler_params=pltpu.CompilerParams(needs_layout_passes=False),
        scratch_types=[
            pltpu.VMEM((N_OBS * D_MODEL,), jnp.float32),
            pltpu.VMEM((N_FEAT * D_MODEL,), jnp.float32),
            pltpu.VMEM((D_MODEL,), jnp.float32),
            pltpu.VMEM((D_MODEL,), jnp.float32),
            pltpu.VMEM((2 * BLKP,), jnp.int32),
            pltpu.VMEM((2 * BLKP,), jnp.int32),
            pltpu.VMEM((2 * BLKP,), jnp.float32),
            pltpu.VMEM((2, BLKP, D_MODEL), jnp.float32),
            pltpu.SemaphoreType.DMA,
            pltpu.SemaphoreType.DMA,
            pltpu.SemaphoreType.DMA,
            pltpu.SemaphoreType.DMA,
        ],
    )
    def embed(val_h, obs_h, feat_h, tab_obs_h, tab_feat_h, w_h, bias_h,
              out_h, tab_o, tab_f, w_v, b_v, obs_v, feat_v, val_v, out_v,
              sem_in0, sem_in1, sem_out0, sem_out1):
        wid = lax.axis_index("s") * NUM_CORES + lax.axis_index("c")
        base = wid * per_w
        row_base = wid * rows_per_w
        sems_in = (sem_in0, sem_in1)
        sems_out = (sem_out0, sem_out1)

        pltpu.sync_copy(tab_obs_h, tab_o)
        pltpu.sync_copy(tab_feat_h, tab_f)
        pltpu.sync_copy(w_h, w_v)
        pltpu.sync_copy(bias_h, b_v)

        w_regs = [w_v[pl.ds(16 * j, 16)] for j in range(NJ)]
        b_regs = [b_v[pl.ds(16 * j, 16)] for j in range(NJ)]

        def bias_body(r, c):
            for j in range(NJ):
                off = r * D_MODEL + 16 * j
                tab_o[pl.ds(off, 16)] = tab_o[pl.ds(off, 16)] + b_regs[j]
            return c

        lax.fori_loop(0, N_OBS, bias_body, 0)

        zeros16i = jnp.zeros((16,), jnp.int32)
        for b in range(2):
            obs_v[pl.ds(b * BLKP + BLKP - 16, 16)] = zeros16i
            feat_v[pl.ds(b * BLKP + BLKP - 16, 16)] = zeros16i
            val_v[pl.ds(b * BLKP + BLKP - 16, 16)] = jnp.zeros(
                (16,), jnp.float32)

        def in_copies(s, b):
            row0 = base + s * BLK
            return (
                pltpu.make_async_copy(obs_h.at[pl.ds(row0, BLK)],
                                      obs_v.at[pl.ds(b * BLKP, BLK)],
                                      sems_in[b]),
                pltpu.make_async_copy(feat_h.at[pl.ds(row0, BLK)],
                                      feat_v.at[pl.ds(b * BLKP, BLK)],
                                      sems_in[b]),
                pltpu.make_async_copy(val_h.at[pl.ds(row0, BLK)],
                                      val_v.at[pl.ds(b * BLKP, BLK)],
                                      sems_in[b]),
            )

        def out_copy(s, b):
            return pltpu.make_async_copy(
                out_v.at[b, pl.ds(0, BLK)],
                out_h.at[row_base + s],
                sems_out[b])

        for c in in_copies(0, 0):
            c.start()

        def pair_body(g, carry):
            for b in range(2):
                s = g * 2 + b

                @pl.when(s + 1 < steps)
                def _():
                    for c in in_copies(s + 1, 1 - b):
                        c.start()

                for c in in_copies(s, b):
                    c.wait()

                @pl.when(s >= 2)
                def _():
                    out_copy(s - 2, b).wait()

                def grp_body(gi, carry2):
                    t0b = b * BLKP + gi * 16
                    o16 = obs_v[pl.ds(t0b, 16)] * D_MODEL
                    f16 = feat_v[pl.ds(t0b, 16)] * D_MODEL
                    v16 = val_v[pl.ds(t0b, 16)]
                    t0 = gi * 16

                    def load_tok(ob, fb):
                        return (
                            [tab_o[pl.ds(ob + 16 * j, 16)] for j in range(NJ)],
                            [tab_f[pl.ds(fb + 16 * j, 16)] for j in range(NJ)],
                        )

                    PF = 2
                    obase = [None] * 16
                    fbase = [None] * 16
                    for k in range(PF):
                        obase[k] = o16[k]
                        fbase[k] = f16[k]
                    ro, rf = load_tok(obase[0], fbase[0])
                    for k in range(16):
                        if k + PF < 16:
                            obase[k + PF] = o16[k + PF]
                            fbase[k + PF] = f16[k + PF]
                        if k + 1 < 16:
                            ro_n, rf_n = load_tok(obase[k + 1], fbase[k + 1])
                        vb = v16[k]
                        for j in range(NJ):
                            out_v[b, t0 + k, pl.ds(16 * j, 16)] = (
                                ro[j] + rf[j] + vb * w_regs[j])
                        if k + 1 < 16:
                            ro, rf = ro_n, rf_n
                    return carry2

                lax.fori_loop(0, BLKP // 16, grp_body, 0)
                out_copy(s, b).start()
            return carry

        lax.fori_loop(0, steps // 2, pair_body, 0)
        out_copy(steps - 2, 0).wait()
        out_copy(steps - 1, 1).wait()

    return embed


def kernel(val, obs_idx, feat_idx, W_val, b_val, emb_obs, emb_feat):
    B, L, _ = val.shape
    T = B * L
    val_f = val.reshape(T).astype(jnp.float32)
    obs_f = obs_idx.reshape(T).astype(jnp.int32)
    feat_f = feat_idx.reshape(T).astype(jnp.int32)
    w_f = W_val.reshape(D_MODEL).astype(jnp.float32)
    b_f = b_val.reshape(D_MODEL).astype(jnp.float32)
    return _build(B, L)(val_f, obs_f, feat_f,
                        emb_obs.astype(jnp.float32).reshape(N_OBS * D_MODEL),
                        emb_feat.astype(jnp.float32).reshape(N_FEAT * D_MODEL),
                        w_f, b_f)
